# baseline (speedup 1.0000x reference)
import functools

import jax
import jax.numpy as jnp
import numpy as np
from jax import lax
from jax.experimental import pallas as pl
from jax.experimental.pallas import tpu as pltpu
from jax.experimental.pallas import tpu_sc as plsc

N = 10000
D = 128
H = 128
E = 320000
ALPHA = 0.1
EPS = 1e-5

NC = 2
NS = 16
NW = NC * NS
CHUNK = 128
CPT = 80
IH = CPT // 2
PADE = NW * CPT * CHUNK
NPAD = 10112
RPS = NPAD // NS

B1 = float(np.log(0.5 / 1 + 1.0))
B2 = float(np.log(0.5 / 2 + 1.0))
B3 = float(np.log(0.5 / 3 + 1.0))

_MESH = plsc.VectorSubcoreMesh(
    core_axis_name="c", subcore_axis_name="s", num_cores=NC, num_subcores=NS
)


def _agg_kernel(x, src2d, dst2d, zeros):
    @functools.partial(
        pl.kernel,
        out_type=jax.ShapeDtypeStruct((NC, NPAD, H), jnp.float32),
        mesh=_MESH,
        scratch_types=[
            pltpu.VMEM((IH, CHUNK), jnp.int32),
            pltpu.VMEM((IH, CHUNK), jnp.int32),
            pltpu.VMEM((CHUNK, H), jnp.float32),
            pltpu.VMEM((CHUNK, H), jnp.float32),
            pltpu.VMEM_SHARED((NPAD, H), jnp.float32),
            pltpu.SemaphoreType.DMA,
            pltpu.SemaphoreType.DMA,
            pltpu.SemaphoreType.DMA,
            pltpu.SemaphoreType.DMA,
            pltpu.SemaphoreType.DMA,
        ],
    )
    def k(x_hbm, src_hbm, dst_hbm, z_hbm, out_hbm,
          src_v, dst_v, rows0, rows1, acc, zsem, g0, g1, s0, s1):
        cid = lax.axis_index("c")
        sid = lax.axis_index("s")
        wid = sid * NC + cid
        base = sid * RPS

        zd = pltpu.async_copy(z_hbm.at[pl.ds(base, RPS)],
                              acc.at[pl.ds(base, RPS)], zsem)

        for half in range(2):
            off = wid * CPT + half * IH
            pltpu.sync_copy(src_hbm.at[pl.ds(off, IH)], src_v)
            pltpu.sync_copy(dst_hbm.at[pl.ds(off, IH)], dst_v)
            if half == 0:
                zd.wait()
                plsc.subcore_barrier()

            @pl.loop(0, IH, step=2)
            def _(j):
                gd0 = pltpu.async_copy(x_hbm.at[src_v.at[j]], rows0, g0)
                gd1 = pltpu.async_copy(x_hbm.at[src_v.at[j + 1]], rows1, g1)
                gd0.wait()
                sd0 = pltpu.async_copy(rows0, acc.at[dst_v.at[j]], s0, add=True)
                gd1.wait()
                sd1 = pltpu.async_copy(rows1, acc.at[dst_v.at[j + 1]], s1,
                                       add=True)
                sd0.wait()
                sd1.wait()

        plsc.subcore_barrier()
        pltpu.sync_copy(acc.at[pl.ds(base, RPS)], out_hbm.at[cid, pl.ds(base, RPS)])

    return k(x, src2d, dst2d, zeros)


def _deg_kernel(dst2d, zeros, ones128):
    @functools.partial(
        pl.kernel,
        out_type=jax.ShapeDtypeStruct((NC, NPAD, H), jnp.float32),
        mesh=_MESH,
        scratch_types=[
            pltpu.VMEM((CPT, CHUNK), jnp.int32),
            pltpu.VMEM((CHUNK, H), jnp.float32),
            pltpu.VMEM_SHARED((NPAD, H), jnp.float32),
            pltpu.SemaphoreType.DMA,
            pltpu.SemaphoreType.DMA,
            pltpu.SemaphoreType.DMA,
        ],
    )
    def k(dst_hbm, z_hbm, o_hbm, out_hbm, dst_v, ones_v, acc, zsem, s0, s1):
        cid = lax.axis_index("c")
        sid = lax.axis_index("s")
        wid = sid * NC + cid
        base = sid * RPS

        zd = pltpu.async_copy(z_hbm.at[pl.ds(base, RPS)],
                              acc.at[pl.ds(base, RPS)], zsem)
        pltpu.sync_copy(o_hbm, ones_v)
        pltpu.sync_copy(dst_hbm.at[pl.ds(wid * CPT, CPT)], dst_v)
        zd.wait()
        plsc.subcore_barrier()

        @pl.loop(0, CPT, step=2)
        def _(j):
            sd0 = pltpu.async_copy(ones_v, acc.at[dst_v.at[j]], s0, add=True)
            sd1 = pltpu.async_copy(ones_v, acc.at[dst_v.at[j + 1]], s1, add=True)
            sd0.wait()
            sd1.wait()

        plsc.subcore_barrier()
        pltpu.sync_copy(acc.at[pl.ds(base, RPS)], out_hbm.at[cid, pl.ds(base, RPS)])

    return k(dst2d, zeros, ones128)


def _dot(a, b):
    return jnp.dot(a, b, preferred_element_type=jnp.float32,
                   precision=lax.Precision.HIGHEST)


def _xw_body(x_ref, w_ref, o_ref):
    o_ref[...] = _dot(x_ref[...], w_ref[...])


def _u_body(xw_ref, degp_ref, o_ref):
    deg = 1.0 + degp_ref[0, :N, 0] + degp_ref[1, :N, 0]
    dinv = lax.rsqrt(deg)
    o_ref[...] = dinv[:, None] * xw_ref[...]


def _z0_body(sp_ref, xw_ref, degp_ref, b0_ref, o_ref):
    deg = 1.0 + degp_ref[0, :N, 0] + degp_ref[1, :N, 0]
    dinv = lax.rsqrt(deg)
    s = sp_ref[0, :N, :] + sp_ref[1, :N, :]
    xw = xw_ref[...]
    o_ref[...] = dinv[:, None] * s + (dinv * dinv)[:, None] * xw + b0_ref[...]


def _layer_body(ap_ref, x0_ref, w_ref, g_ref, be_ref, o_ref, *, beta, bn):
    agg = ap_ref[0, :N, :] + ap_ref[1, :N, :]
    h = (1.0 - ALPHA) * agg + ALPHA * x0_ref[...]
    z = (1.0 - beta) * h + beta * _dot(h, w_ref[...])
    if bn:
        m = jnp.mean(z, axis=0)
        v = jnp.mean(z * z, axis=0) - m * m
        z = (z - m) * lax.rsqrt(v + EPS) * g_ref[...] + be_ref[...]
        z = jnp.maximum(z, 0.0)
    o_ref[...] = z


def _tc(body, *args):
    return pl.pallas_call(
        body, out_shape=jax.ShapeDtypeStruct((N, H), jnp.float32)
    )(*args)


def kernel(x, edge_index, W0, b0, W1_1, W1_2, W1_3, g0, be0, g1, be1):
    pad = PADE - E
    pad_src = jnp.arange(pad, dtype=jnp.int32) % N
    src = jnp.concatenate([edge_index[0], pad_src])
    pad_dst = N + (jnp.arange(pad, dtype=jnp.int32) % (NPAD - N))
    dst = jnp.concatenate([edge_index[1], pad_dst])
    src2d = src.reshape(NW * CPT, CHUNK)
    dst2d = dst.reshape(NW * CPT, CHUNK)
    zeros = jnp.zeros((NPAD, H), jnp.float32)

    ones128 = jnp.ones((CHUNK, H), jnp.float32)
    degp = _deg_kernel(dst2d, zeros, ones128)
    xw = _tc(_xw_body, x, W0)
    u = _tc(_u_body, xw, degp)
    sp = _agg_kernel(u, src2d, dst2d, zeros)
    x0 = _tc(_z0_body, sp, xw, degp, b0)

    ap = _agg_kernel(x0, src2d, dst2d, zeros)
    z = _tc(functools.partial(_layer_body, beta=B1, bn=True), ap, x0, W1_1, g0, be0)
    ap = _agg_kernel(z, src2d, dst2d, zeros)
    z = _tc(functools.partial(_layer_body, beta=B2, bn=True), ap, x0, W1_2, g1, be1)
    ap = _agg_kernel(z, src2d, dst2d, zeros)
    z = _tc(functools.partial(_layer_body, beta=B3, bn=False), ap, x0, W1_3, g1, be1)
    return z


# speedup vs baseline: 1.1037x; 1.1037x over previous
import functools

import jax
import jax.numpy as jnp
import numpy as np
from jax import lax
from jax.experimental import pallas as pl
from jax.experimental.pallas import tpu as pltpu
from jax.experimental.pallas import tpu_sc as plsc

N = 10000
D = 128
H = 128
E = 320000
ALPHA = 0.1
EPS = 1e-5

NC = 2
NS = 16
NW = NC * NS
CHUNK = 128
CPT = 80
IH = CPT // 2
PADE = NW * CPT * CHUNK
NPAD = 10112
RPS = NPAD // NS

B1 = float(np.log(0.5 / 1 + 1.0))
B2 = float(np.log(0.5 / 2 + 1.0))
B3 = float(np.log(0.5 / 3 + 1.0))

_MESH = plsc.VectorSubcoreMesh(
    core_axis_name="c", subcore_axis_name="s", num_cores=NC, num_subcores=NS
)


def _agg_kernel(x, src2d, dst2d, zeros):
    @functools.partial(
        pl.kernel,
        out_type=jax.ShapeDtypeStruct((NC, NPAD, H), jnp.float32),
        mesh=_MESH,
        scratch_types=[
            pltpu.VMEM((IH, CHUNK), jnp.int32),
            pltpu.VMEM((IH, CHUNK), jnp.int32),
            pltpu.VMEM((CHUNK, H), jnp.float32),
            pltpu.VMEM((CHUNK, H), jnp.float32),
            pltpu.VMEM_SHARED((NPAD, H), jnp.float32),
            pltpu.SemaphoreType.DMA,
            pltpu.SemaphoreType.DMA,
            pltpu.SemaphoreType.DMA,
            pltpu.SemaphoreType.DMA,
            pltpu.SemaphoreType.DMA,
        ],
    )
    def k(x_hbm, src_hbm, dst_hbm, z_hbm, out_hbm,
          src_v, dst_v, rows0, rows1, acc, zsem, g0, g1, s0, s1):
        cid = lax.axis_index("c")
        sid = lax.axis_index("s")
        wid = sid * NC + cid
        base = sid * RPS

        zd = pltpu.async_copy(z_hbm.at[pl.ds(base, RPS)],
                              acc.at[pl.ds(base, RPS)], zsem)

        for half in range(2):
            off = wid * CPT + half * IH
            pltpu.sync_copy(src_hbm.at[pl.ds(off, IH)], src_v)
            pltpu.sync_copy(dst_hbm.at[pl.ds(off, IH)], dst_v)
            if half == 0:
                zd.wait()
                plsc.subcore_barrier()

            pltpu.async_copy(x_hbm.at[src_v.at[0]], rows0, g0)

            @pl.loop(0, IH, step=2)
            def _(j):
                pltpu.make_async_copy(x_hbm.at[src_v.at[j]], rows0, g0).wait()
                sd0 = pltpu.async_copy(rows0, acc.at[dst_v.at[j]], s0, add=True)

                @pl.when(j > 0)
                def _():
                    pltpu.make_async_copy(rows1, acc.at[dst_v.at[j]], s1).wait()

                pltpu.async_copy(x_hbm.at[src_v.at[j + 1]], rows1, g1).wait()
                pltpu.async_copy(rows1, acc.at[dst_v.at[j + 1]], s1, add=True)

                @pl.when(j + 2 < IH)
                def _():
                    sd0.wait()
                    pltpu.async_copy(x_hbm.at[src_v.at[j + 2]], rows0, g0)

            pltpu.make_async_copy(rows0, acc.at[dst_v.at[0]], s0).wait()
            pltpu.make_async_copy(rows1, acc.at[dst_v.at[0]], s1).wait()

        plsc.subcore_barrier()
        pltpu.sync_copy(acc.at[pl.ds(base, RPS)], out_hbm.at[cid, pl.ds(base, RPS)])

    return k(x, src2d, dst2d, zeros)


def _deg_kernel(dst2d, zeros, ones128):
    @functools.partial(
        pl.kernel,
        out_type=jax.ShapeDtypeStruct((NC, NPAD, H), jnp.float32),
        mesh=_MESH,
        scratch_types=[
            pltpu.VMEM((CPT, CHUNK), jnp.int32),
            pltpu.VMEM((CHUNK, H), jnp.float32),
            pltpu.VMEM_SHARED((NPAD, H), jnp.float32),
            pltpu.SemaphoreType.DMA,
            pltpu.SemaphoreType.DMA,
            pltpu.SemaphoreType.DMA,
        ],
    )
    def k(dst_hbm, z_hbm, o_hbm, out_hbm, dst_v, ones_v, acc, zsem, s0, s1):
        cid = lax.axis_index("c")
        sid = lax.axis_index("s")
        wid = sid * NC + cid
        base = sid * RPS

        zd = pltpu.async_copy(z_hbm.at[pl.ds(base, RPS)],
                              acc.at[pl.ds(base, RPS)], zsem)
        pltpu.sync_copy(o_hbm, ones_v)
        pltpu.sync_copy(dst_hbm.at[pl.ds(wid * CPT, CPT)], dst_v)
        zd.wait()
        plsc.subcore_barrier()

        @pl.loop(0, CPT, step=2)
        def _(j):
            sd0 = pltpu.async_copy(ones_v, acc.at[dst_v.at[j]], s0, add=True)
            sd1 = pltpu.async_copy(ones_v, acc.at[dst_v.at[j + 1]], s1, add=True)
            sd0.wait()
            sd1.wait()

        plsc.subcore_barrier()
        pltpu.sync_copy(acc.at[pl.ds(base, RPS)], out_hbm.at[cid, pl.ds(base, RPS)])

    return k(dst2d, zeros, ones128)


def _dot(a, b):
    return jnp.dot(a, b, preferred_element_type=jnp.float32,
                   precision=lax.Precision.HIGHEST)


def _xw_body(x_ref, w_ref, o_ref):
    o_ref[...] = _dot(x_ref[...], w_ref[...])


def _u_body(xw_ref, degp_ref, o_ref):
    deg = 1.0 + degp_ref[0, :N, 0] + degp_ref[1, :N, 0]
    dinv = lax.rsqrt(deg)
    o_ref[...] = dinv[:, None] * xw_ref[...]


def _z0_body(sp_ref, xw_ref, degp_ref, b0_ref, o_ref):
    deg = 1.0 + degp_ref[0, :N, 0] + degp_ref[1, :N, 0]
    dinv = lax.rsqrt(deg)
    s = sp_ref[0, :N, :] + sp_ref[1, :N, :]
    xw = xw_ref[...]
    o_ref[...] = dinv[:, None] * s + (dinv * dinv)[:, None] * xw + b0_ref[...]


def _layer_body(ap_ref, x0_ref, w_ref, g_ref, be_ref, o_ref, *, beta, bn):
    agg = ap_ref[0, :N, :] + ap_ref[1, :N, :]
    h = (1.0 - ALPHA) * agg + ALPHA * x0_ref[...]
    z = (1.0 - beta) * h + beta * _dot(h, w_ref[...])
    if bn:
        m = jnp.mean(z, axis=0)
        v = jnp.mean(z * z, axis=0) - m * m
        z = (z - m) * lax.rsqrt(v + EPS) * g_ref[...] + be_ref[...]
        z = jnp.maximum(z, 0.0)
    o_ref[...] = z


def _tc(body, *args):
    return pl.pallas_call(
        body, out_shape=jax.ShapeDtypeStruct((N, H), jnp.float32)
    )(*args)


def kernel(x, edge_index, W0, b0, W1_1, W1_2, W1_3, g0, be0, g1, be1):
    pad = PADE - E
    pad_src = jnp.arange(pad, dtype=jnp.int32) % N
    src = jnp.concatenate([edge_index[0], pad_src])
    pad_dst = N + (jnp.arange(pad, dtype=jnp.int32) % (NPAD - N))
    dst = jnp.concatenate([edge_index[1], pad_dst])
    src2d = src.reshape(NW * CPT, CHUNK)
    dst2d = dst.reshape(NW * CPT, CHUNK)
    zeros = jnp.zeros((NPAD, H), jnp.float32)

    ones128 = jnp.ones((CHUNK, H), jnp.float32)
    degp = _deg_kernel(dst2d, zeros, ones128)
    xw = _tc(_xw_body, x, W0)
    u = _tc(_u_body, xw, degp)
    sp = _agg_kernel(u, src2d, dst2d, zeros)
    x0 = _tc(_z0_body, sp, xw, degp, b0)

    ap = _agg_kernel(x0, src2d, dst2d, zeros)
    z = _tc(functools.partial(_layer_body, beta=B1, bn=True), ap, x0, W1_1, g0, be0)
    ap = _agg_kernel(z, src2d, dst2d, zeros)
    z = _tc(functools.partial(_layer_body, beta=B2, bn=True), ap, x0, W1_2, g1, be1)
    ap = _agg_kernel(z, src2d, dst2d, zeros)
    z = _tc(functools.partial(_layer_body, beta=B3, bn=False), ap, x0, W1_3, g1, be1)
    return z
